# baseline (speedup 1.0000x reference)
import jax
import jax.numpy as jnp
from jax import lax
from jax.experimental import pallas as pl
from jax.experimental.pallas import tpu as pltpu
from jax.experimental.pallas import tpu_sc as plsc

_VOCAB = 4096
_N = 8192
_NC = 2
_NS = 16
_NW = _NC * _NS
_BPW = _N // _NW
_C = 8
_NB = 3
_NCHUNK = _BPW // _C
_L = 16

_TRB = 512


def _lse_body(table_ref, lse_ref):
    blk = table_ref[...]
    mx = jnp.max(blk, axis=1, keepdims=True)
    se = jnp.sum(jnp.exp(blk - mx), axis=1, keepdims=True)
    lse_ref[...] = mx + jnp.log(se)


def _sc_gather_body(
    table_ref,
    idx_ref,
    tgt_ref,
    out_ref,
    part_ref,
    hist_ref,
    idx_v,
    tgt_v,
    part_v,
    counts_v,
    rows0,
    rows1,
    rows2,
    gsem0,
    gsem1,
    gsem2,
    osem0,
    osem1,
    osem2,
):
    c = lax.axis_index("c")
    s = lax.axis_index("s")
    wid = s * _NC + c
    base = wid * _BPW
    bi = wid // (idx_ref.shape[1] // _BPW)
    off = (wid % (idx_ref.shape[1] // _BPW)) * _BPW

    pltpu.sync_copy(idx_ref.at[bi, pl.ds(off, _BPW)], idx_v)
    pltpu.sync_copy(tgt_ref.at[bi, pl.ds(off, _BPW)], tgt_v.at[pl.ds(0, _BPW)])

    rows = (rows0, rows1, rows2)
    gsem = (gsem0, gsem1, gsem2)
    osem = (osem0, osem1, osem2)
    g_cp = [None] * _NB
    o_cp = [None] * _NB

    lanes = lax.broadcasted_iota(jnp.int32, (_L,), 0)
    row_idx = lanes & (_C - 1)
    lo_mask = lanes < _C
    zero = jnp.zeros((_L,), jnp.float32)
    acc_pick = zero

    def start_gather(k):
        b = k % _NB
        g_cp[b] = pltpu.async_copy(
            table_ref.at[idx_v.at[pl.ds(k * _C, _C)]], rows[b], gsem[b]
        )

    for k in range(_NB):
        start_gather(k)

    for j in range(_NCHUNK):
        b = j % _NB
        g_cp[b].wait()
        col = tgt_v[pl.ds(j * _C, _L)] & (_VOCAB - 1)
        g = plsc.load_gather(rows[b], [row_idx, col], mask=lo_mask)
        acc_pick = acc_pick + jnp.where(lo_mask, g, zero)
        o_cp[b] = pltpu.async_copy(
            rows[b], out_ref.at[pl.ds(base + j * _C, _C)], osem[b]
        )
        k = j + _NB
        if k < _NCHUNK:
            o_cp[b].wait()
            start_gather(k)
    for cp in o_cp:
        cp.wait()
    part_v[...] = acc_pick
    pltpu.sync_copy(part_v, part_ref.at[pl.ds(wid * _L, _L)])

    for j in range(_VOCAB // _L):
        counts_v[pl.ds(j * _L, _L)] = zero
    ones = jnp.ones((_L,), jnp.float32)
    for j in range(_BPW // _L):
        iv = idx_v[pl.ds(j * _L, _L)]
        plsc.addupdate_scatter(counts_v, [iv], ones)
    pltpu.sync_copy(counts_v, hist_ref.at[wid])


def _combine_body(hist_ref, lse_ref, pick_ref, loss_ref):
    tot = jnp.sum(
        jax.lax.dot_general(
            hist_ref[...], lse_ref[...], (((1,), (0,)), ((), ())),
            preferred_element_type=jnp.float32,
            precision=jax.lax.Precision.HIGHEST,
        )
    )
    loss_ref[...] = ((tot - jnp.sum(pick_ref[...])) / _N).reshape(1, 1)


@jax.jit
def kernel(index, target, table):
    b, s = index.shape
    v = table.shape[1]

    mesh = plsc.VectorSubcoreMesh(core_axis_name="c", subcore_axis_name="s")
    sc_gather = pl.kernel(
        _sc_gather_body,
        out_type=(
            jax.ShapeDtypeStruct((_N, v), jnp.float32),
            jax.ShapeDtypeStruct((_NW * _L,), jnp.float32),
            jax.ShapeDtypeStruct((_NW, _VOCAB), jnp.float32),
        ),
        mesh=mesh,
        compiler_params=pltpu.CompilerParams(needs_layout_passes=False),
        scratch_types=[
            pltpu.VMEM((_BPW,), jnp.int32),
            pltpu.VMEM((_BPW + _L,), jnp.int32),
            pltpu.VMEM((_L,), jnp.float32),
            pltpu.VMEM((_VOCAB,), jnp.float32),
            pltpu.VMEM((_C, _VOCAB), jnp.float32),
            pltpu.VMEM((_C, _VOCAB), jnp.float32),
            pltpu.VMEM((_C, _VOCAB), jnp.float32),
            pltpu.SemaphoreType.DMA,
            pltpu.SemaphoreType.DMA,
            pltpu.SemaphoreType.DMA,
            pltpu.SemaphoreType.DMA,
            pltpu.SemaphoreType.DMA,
            pltpu.SemaphoreType.DMA,
        ],
    )
    logits_flat, pick_parts, hist = sc_gather(table, index, target)

    lse = pl.pallas_call(
        _lse_body,
        grid=(_VOCAB // _TRB,),
        in_specs=[pl.BlockSpec((_TRB, v), lambda i: (i, 0))],
        out_specs=pl.BlockSpec((_TRB, 1), lambda i: (i, 0)),
        out_shape=jax.ShapeDtypeStruct((_VOCAB, 1), jnp.float32),
    )(table)

    loss = pl.pallas_call(
        _combine_body,
        out_shape=jax.ShapeDtypeStruct((1, 1), jnp.float32),
    )(hist, lse, pick_parts.reshape(1, _NW * _L))

    return logits_flat.reshape(b, s, v), loss[0, 0]


# speedup vs baseline: 1.0118x; 1.0118x over previous
import jax
import jax.numpy as jnp
from jax import lax
from jax.experimental import pallas as pl
from jax.experimental.pallas import tpu as pltpu
from jax.experimental.pallas import tpu_sc as plsc

_VOCAB = 4096
_N = 8192
_NC = 2
_NS = 16
_NW = _NC * _NS
_BPW = _N // _NW
_C = 8
_NB = 3
_NCHUNK = _BPW // _C
_L = 16

_TRB = 512


def _lse_body(table_ref, lse_ref):
    blk = table_ref[...]
    mx = jnp.max(blk, axis=1, keepdims=True)
    se = jnp.sum(jnp.exp(blk - mx), axis=1, keepdims=True)
    lse_ref[...] = mx + jnp.log(se)


def _sc_gather_body(
    table_ref,
    idx_ref,
    tgt_ref,
    out_ref,
    part_ref,
    hist_ref,
    idx_v,
    tgt_v,
    part_v,
    counts_v,
    rows0,
    rows1,
    rows2,
    gsem0,
    gsem1,
    gsem2,
    osem0,
    osem1,
    osem2,
):
    c = lax.axis_index("c")
    s = lax.axis_index("s")
    wid = s * _NC + c
    base = wid * _BPW
    bi = wid // (idx_ref.shape[1] // _BPW)
    off = (wid % (idx_ref.shape[1] // _BPW)) * _BPW

    pltpu.sync_copy(idx_ref.at[bi, pl.ds(off, _BPW)], idx_v)
    pltpu.sync_copy(tgt_ref.at[bi, pl.ds(off, _BPW)], tgt_v.at[pl.ds(0, _BPW)])

    rows = (rows0, rows1, rows2)
    gsem = (gsem0, gsem1, gsem2)
    osem = (osem0, osem1, osem2)
    g_cp = [None] * _NB
    o_cp = [None] * _NB

    lanes = lax.broadcasted_iota(jnp.int32, (_L,), 0)
    row_idx = lanes & (_C - 1)
    lo_mask = lanes < _C
    zero = jnp.zeros((_L,), jnp.float32)
    acc_pick = zero

    def start_gather(k):
        b = k % _NB
        g_cp[b] = pltpu.async_copy(
            table_ref.at[idx_v.at[pl.ds(k * _C, _C)]], rows[b], gsem[b]
        )

    for k in range(_NB):
        start_gather(k)

    for j in range(_NCHUNK):
        b = j % _NB
        g_cp[b].wait()
        o_cp[b] = pltpu.async_copy(
            rows[b], out_ref.at[pl.ds(base + j * _C, _C)], osem[b]
        )
        col = tgt_v[pl.ds(j * _C, _L)] & (_VOCAB - 1)
        g = plsc.load_gather(rows[b], [row_idx, col], mask=lo_mask)
        acc_pick = acc_pick + jnp.where(lo_mask, g, zero)
        k = j + _NB
        if k < _NCHUNK:
            o_cp[b].wait()
            start_gather(k)
    for cp in o_cp:
        cp.wait()
    part_v[...] = acc_pick
    pltpu.sync_copy(part_v, part_ref.at[pl.ds(wid * _L, _L)])

    for j in range(_VOCAB // _L):
        counts_v[pl.ds(j * _L, _L)] = zero
    ones = jnp.ones((_L,), jnp.float32)
    for j in range(_BPW // _L):
        iv = idx_v[pl.ds(j * _L, _L)]
        plsc.addupdate_scatter(counts_v, [iv], ones)
    pltpu.sync_copy(counts_v, hist_ref.at[wid])


def _combine_body(hist_ref, lse_ref, pick_ref, loss_ref):
    tot = jnp.sum(
        jax.lax.dot_general(
            hist_ref[...], lse_ref[...], (((1,), (0,)), ((), ())),
            preferred_element_type=jnp.float32,
            precision=jax.lax.Precision.HIGHEST,
        )
    )
    loss_ref[...] = ((tot - jnp.sum(pick_ref[...])) / _N).reshape(1, 1)


@jax.jit
def kernel(index, target, table):
    b, s = index.shape
    v = table.shape[1]

    mesh = plsc.VectorSubcoreMesh(core_axis_name="c", subcore_axis_name="s")
    sc_gather = pl.kernel(
        _sc_gather_body,
        out_type=(
            jax.ShapeDtypeStruct((_N, v), jnp.float32),
            jax.ShapeDtypeStruct((_NW * _L,), jnp.float32),
            jax.ShapeDtypeStruct((_NW, _VOCAB), jnp.float32),
        ),
        mesh=mesh,
        compiler_params=pltpu.CompilerParams(needs_layout_passes=False),
        scratch_types=[
            pltpu.VMEM((_BPW,), jnp.int32),
            pltpu.VMEM((_BPW + _L,), jnp.int32),
            pltpu.VMEM((_L,), jnp.float32),
            pltpu.VMEM((_VOCAB,), jnp.float32),
            pltpu.VMEM((_C, _VOCAB), jnp.float32),
            pltpu.VMEM((_C, _VOCAB), jnp.float32),
            pltpu.VMEM((_C, _VOCAB), jnp.float32),
            pltpu.SemaphoreType.DMA,
            pltpu.SemaphoreType.DMA,
            pltpu.SemaphoreType.DMA,
            pltpu.SemaphoreType.DMA,
            pltpu.SemaphoreType.DMA,
            pltpu.SemaphoreType.DMA,
        ],
    )
    logits_flat, pick_parts, hist = sc_gather(table, index, target)

    lse = pl.pallas_call(
        _lse_body,
        grid=(_VOCAB // _TRB,),
        in_specs=[pl.BlockSpec((_TRB, v), lambda i: (i, 0))],
        out_specs=pl.BlockSpec((_TRB, 1), lambda i: (i, 0)),
        out_shape=jax.ShapeDtypeStruct((_VOCAB, 1), jnp.float32),
    )(table)

    loss = pl.pallas_call(
        _combine_body,
        out_shape=jax.ShapeDtypeStruct((1, 1), jnp.float32),
    )(hist, lse, pick_parts.reshape(1, _NW * _L))

    return logits_flat.reshape(b, s, v), loss[0, 0]
